# baseline (speedup 1.0000x reference)
import jax
import jax.numpy as jnp
from jax.experimental import pallas as pl
from jax.experimental.pallas import tpu as pltpu

_BNB = 200
_BN = 200
_TI = 200


def _boxes_body(mask_ref, lab_ref, out_ref):
    m = mask_ref[...]
    bn, h, w = m.shape
    proj_x = jnp.max(m, axis=1) > 0.5
    proj_y = jnp.max(m, axis=2) > 0.5
    xs = jax.lax.broadcasted_iota(jnp.int32, (bn, w), 1).astype(jnp.float32)
    ys = jax.lax.broadcasted_iota(jnp.int32, (bn, h), 1).astype(jnp.float32)
    off = lab_ref[...] * 128.0
    x1 = jnp.min(jnp.where(proj_x, xs, float(w)), axis=1, keepdims=True)
    x2 = jnp.max(jnp.where(proj_x, xs, -1.0), axis=1, keepdims=True)
    y1 = jnp.min(jnp.where(proj_y, ys, float(h)), axis=1, keepdims=True)
    y2 = jnp.max(jnp.where(proj_y, ys, -1.0), axis=1, keepdims=True)
    out_ref[:, 0:1] = x1 + off
    out_ref[:, 1:2] = y1
    out_ref[:, 2:3] = x2 + off
    out_ref[:, 3:4] = y2


def _nms_body(bc_ref, scc_ref, scr_ref, pts_ref,
              keep_ref, opts_ref,
              kc_ref, nkc_ref, srow_ref, rkc_ref, rkr_ref, rec_ref, rer_ref,
              br_ref):
    n = bc_ref.shape[0]
    br_ref[...] = jnp.transpose(bc_ref[...], (1, 0))
    nt = n // _TI
    big = 3.0e7

    rkr_ref[...] = jnp.zeros((1, n), jnp.float32)
    scr = scr_ref[...]
    idxr = jax.lax.broadcasted_iota(jnp.int32, (1, n), 1).astype(jnp.float32)

    def rtile(t, _):
        a = pl.ds(t * _TI, _TI)
        sa = scc_ref[a, :]
        ia = (jax.lax.broadcasted_iota(jnp.int32, (_TI, 1), 0)
              + t * _TI).astype(jnp.float32)
        prec_ba = (scr > sa) | ((scr == sa) & (idxr < ia))
        rkc_ref[a, :] = jnp.sum(prec_ba.astype(jnp.float32), axis=1,
                                keepdims=True)
        prec_ab = (~prec_ba) & (idxr != ia)
        rkr_ref[...] += jnp.sum(prec_ab.astype(jnp.float32), axis=0,
                                keepdims=True)
        return 0

    jax.lax.fori_loop(0, nt, rtile, 0)

    kc_ref[...] = jnp.ones((n, 1), jnp.float32)
    rec_ref[...] = rkc_ref[...]
    rer_ref[...] = rkr_ref[...]

    def cond(st):
        return st[0] & (st[1] < n)

    def body(st):
        _, it = st
        srow_ref[...] = jnp.zeros((1, n), jnp.float32)
        x1r = br_ref[0:1, :]
        y1r = br_ref[1:2, :]
        x2r = br_ref[2:3, :]
        y2r = br_ref[3:4, :]
        arear4 = 4.0 * (x2r - x1r) * (y2r - y1r)
        rer = rer_ref[...]
        rkrow = rkr_ref[...]

        def tile(t, _):
            a = pl.ds(t * _TI, _TI)
            x1a = bc_ref[a, 0:1]
            y1a = bc_ref[a, 1:2]
            x2a = bc_ref[a, 2:3]
            y2a = bc_ref[a, 3:4]
            ix1 = jnp.maximum(x1a, x1r)
            iy1 = jnp.maximum(y1a, y1r)
            ix2 = jnp.minimum(x2a, x2r)
            iy2 = jnp.minimum(y2a, y2r)
            inter = jnp.maximum(ix2 - ix1, 0.0) * jnp.maximum(iy2 - iy1, 0.0)
            areaa4 = 4.0 * (x2a - x1a) * (y2a - y1a)
            over = 9.0 * inter > areaa4 + arear4
            sup_a = jnp.any(over & (rer < rkc_ref[a, :]), axis=1,
                            keepdims=True)
            nkc_ref[a, :] = 1.0 - sup_a.astype(jnp.float32)
            sup_b = jnp.any(over & (rec_ref[a, :] < rkrow), axis=0,
                            keepdims=True)
            srow_ref[...] = jnp.maximum(srow_ref[...],
                                        sup_b.astype(jnp.float32))
            return 0

        jax.lax.fori_loop(0, nt, tile, 0)
        nkc = nkc_ref[...]
        changed = jnp.any(nkc != kc_ref[...])
        kc_ref[...] = nkc
        rec_ref[...] = jnp.where(nkc > 0.0, rkc_ref[...], big)
        rer_ref[...] = jnp.where(srow_ref[...] > 0.0, big, rkr_ref[...])
        return changed, it + 1

    jax.lax.while_loop(cond, body, (jnp.bool_(True), jnp.int32(0)))
    kf = kc_ref[...]
    keep_ref[...] = kf
    opts_ref[...] = pts_ref[...] * kf


def _apply_body(mask_ref, keep_ref, out_ref):
    out_ref[...] = mask_ref[...] * keep_ref[...]


def kernel(masks, scores, foreground_points, labels):
    n, h, w = masks.shape
    masks = masks.astype(jnp.float32)

    lab_f = labels.astype(jnp.float32)
    boxes = pl.pallas_call(
        _boxes_body,
        grid=(n // _BNB,),
        in_specs=[
            pl.BlockSpec((_BNB, h, w), lambda i: (i, 0, 0)),
            pl.BlockSpec((_BNB, 1), lambda i: (i, 0)),
        ],
        out_specs=pl.BlockSpec((_BNB, 4), lambda i: (i, 0)),
        out_shape=jax.ShapeDtypeStruct((n, 4), jnp.float32),
        compiler_params=pltpu.CompilerParams(
            dimension_semantics=("arbitrary",)),
    )(masks, lab_f.reshape(n, 1))

    keep, kept_points = pl.pallas_call(
        _nms_body,
        out_shape=[
            jax.ShapeDtypeStruct((n, 1), jnp.float32),
            jax.ShapeDtypeStruct((n, 3), jnp.float32),
        ],
        scratch_shapes=[
            pltpu.VMEM((n, 1), jnp.float32),
            pltpu.VMEM((n, 1), jnp.float32),
            pltpu.VMEM((1, n), jnp.float32),
            pltpu.VMEM((n, 1), jnp.float32),
            pltpu.VMEM((1, n), jnp.float32),
            pltpu.VMEM((n, 1), jnp.float32),
            pltpu.VMEM((1, n), jnp.float32),
            pltpu.VMEM((4, n), jnp.float32),
        ],
    )(boxes, scores.reshape(n, 1), scores.reshape(1, n),
      foreground_points.astype(jnp.float32))

    kept_masks = pl.pallas_call(
        _apply_body,
        grid=(n // _BN,),
        in_specs=[
            pl.BlockSpec((_BN, h, w), lambda i: (i, 0, 0)),
            pl.BlockSpec((_BN, 1, 1), lambda i: (i, 0, 0)),
        ],
        out_specs=pl.BlockSpec((_BN, h, w), lambda i: (i, 0, 0)),
        out_shape=jax.ShapeDtypeStruct((n, h, w), jnp.float32),
        compiler_params=pltpu.CompilerParams(
            dimension_semantics=("arbitrary",)),
    )(masks, keep.reshape(n, 1, 1))

    return kept_masks, kept_points.astype(foreground_points.dtype)


# speedup vs baseline: 1.0159x; 1.0159x over previous
import jax
import jax.numpy as jnp
from jax.experimental import pallas as pl
from jax.experimental.pallas import tpu as pltpu

_BNB = 200
_BN = 200
_TI = 200


def _boxes_body(mask_ref, lab_ref, out_ref):
    m = mask_ref[...]
    bn, h, w = m.shape
    proj_x = jnp.max(m, axis=1) > 0.5
    proj_y = jnp.max(m, axis=2) > 0.5
    xs = jax.lax.broadcasted_iota(jnp.int32, (bn, w), 1).astype(jnp.float32)
    ys = jax.lax.broadcasted_iota(jnp.int32, (bn, h), 1).astype(jnp.float32)
    off = lab_ref[...] * 128.0
    x1 = jnp.min(jnp.where(proj_x, xs, float(w)), axis=1, keepdims=True)
    x2 = jnp.max(jnp.where(proj_x, xs, -1.0), axis=1, keepdims=True)
    y1 = jnp.min(jnp.where(proj_y, ys, float(h)), axis=1, keepdims=True)
    y2 = jnp.max(jnp.where(proj_y, ys, -1.0), axis=1, keepdims=True)
    out_ref[:, 0:1] = x1 + off
    out_ref[:, 1:2] = y1
    out_ref[:, 2:3] = x2 + off
    out_ref[:, 3:4] = y2


def _nms_body(bc_ref, scc_ref, scr_ref, pts_ref,
              keep_ref, opts_ref,
              kc_ref, nkc_ref, srow_ref, rkc_ref, rkr_ref, rec_ref, rer_ref,
              br_ref):
    n = bc_ref.shape[0]
    br_ref[...] = jnp.transpose(bc_ref[...], (1, 0))
    nt = n // _TI
    big = 3.0e7

    rkr_ref[...] = jnp.zeros((1, n), jnp.float32)
    srow_ref[...] = jnp.zeros((1, n), jnp.float32)
    scr = scr_ref[...]
    idxr = jax.lax.broadcasted_iota(jnp.int32, (1, n), 1).astype(jnp.float32)
    x1r0 = br_ref[0:1, :]
    y1r0 = br_ref[1:2, :]
    x2r0 = br_ref[2:3, :]
    y2r0 = br_ref[3:4, :]
    arear40 = 4.0 * (x2r0 - x1r0) * (y2r0 - y1r0)

    def rtile(t, _):
        a = pl.ds(t * _TI, _TI)
        x1a = bc_ref[a, 0:1]
        y1a = bc_ref[a, 1:2]
        x2a = bc_ref[a, 2:3]
        y2a = bc_ref[a, 3:4]
        ix1 = jnp.maximum(x1a, x1r0)
        iy1 = jnp.maximum(y1a, y1r0)
        ix2 = jnp.minimum(x2a, x2r0)
        iy2 = jnp.minimum(y2a, y2r0)
        inter = jnp.maximum(ix2 - ix1, 0.0) * jnp.maximum(iy2 - iy1, 0.0)
        areaa4 = 4.0 * (x2a - x1a) * (y2a - y1a)
        over = 9.0 * inter > areaa4 + arear40
        sa = scc_ref[a, :]
        ia = (jax.lax.broadcasted_iota(jnp.int32, (_TI, 1), 0)
              + t * _TI).astype(jnp.float32)
        prec_ba = (scr > sa) | ((scr == sa) & (idxr < ia))
        rkc_ref[a, :] = jnp.sum(prec_ba.astype(jnp.float32), axis=1,
                                keepdims=True)
        sup_a = jnp.any(over & prec_ba, axis=1, keepdims=True)
        nkc_ref[a, :] = 1.0 - sup_a.astype(jnp.float32)
        prec_ab = (~prec_ba) & (idxr != ia)
        rkr_ref[...] += jnp.sum(prec_ab.astype(jnp.float32), axis=0,
                                keepdims=True)
        sup_b = jnp.any(over & prec_ab, axis=0, keepdims=True)
        srow_ref[...] = jnp.maximum(srow_ref[...], sup_b.astype(jnp.float32))
        return 0

    jax.lax.fori_loop(0, nt, rtile, 0)

    kc_ref[...] = nkc_ref[...]
    rec_ref[...] = jnp.where(nkc_ref[...] > 0.0, rkc_ref[...], big)
    rer_ref[...] = jnp.where(srow_ref[...] > 0.0, big, rkr_ref[...])

    def cond(st):
        return st[0] & (st[1] < n)

    def body(st):
        _, it = st
        srow_ref[...] = jnp.zeros((1, n), jnp.float32)
        x1r = br_ref[0:1, :]
        y1r = br_ref[1:2, :]
        x2r = br_ref[2:3, :]
        y2r = br_ref[3:4, :]
        arear4 = 4.0 * (x2r - x1r) * (y2r - y1r)
        rer = rer_ref[...]
        rkrow = rkr_ref[...]

        def tile(t, _):
            a = pl.ds(t * _TI, _TI)
            x1a = bc_ref[a, 0:1]
            y1a = bc_ref[a, 1:2]
            x2a = bc_ref[a, 2:3]
            y2a = bc_ref[a, 3:4]
            ix1 = jnp.maximum(x1a, x1r)
            iy1 = jnp.maximum(y1a, y1r)
            ix2 = jnp.minimum(x2a, x2r)
            iy2 = jnp.minimum(y2a, y2r)
            inter = jnp.maximum(ix2 - ix1, 0.0) * jnp.maximum(iy2 - iy1, 0.0)
            areaa4 = 4.0 * (x2a - x1a) * (y2a - y1a)
            over = 9.0 * inter > areaa4 + arear4
            sup_a = jnp.any(over & (rer < rkc_ref[a, :]), axis=1,
                            keepdims=True)
            nkc_ref[a, :] = 1.0 - sup_a.astype(jnp.float32)
            sup_b = jnp.any(over & (rec_ref[a, :] < rkrow), axis=0,
                            keepdims=True)
            srow_ref[...] = jnp.maximum(srow_ref[...],
                                        sup_b.astype(jnp.float32))
            return 0

        jax.lax.fori_loop(0, nt, tile, 0)
        nkc = nkc_ref[...]
        changed = jnp.any(nkc != kc_ref[...])
        kc_ref[...] = nkc
        rec_ref[...] = jnp.where(nkc > 0.0, rkc_ref[...], big)
        rer_ref[...] = jnp.where(srow_ref[...] > 0.0, big, rkr_ref[...])
        return changed, it + 1

    jax.lax.while_loop(cond, body, (jnp.bool_(True), jnp.int32(0)))
    kf = kc_ref[...]
    keep_ref[...] = kf
    opts_ref[...] = pts_ref[...] * kf


def _apply_body(mask_ref, keep_ref, out_ref):
    out_ref[...] = mask_ref[...] * keep_ref[...]


def kernel(masks, scores, foreground_points, labels):
    n, h, w = masks.shape
    masks = masks.astype(jnp.float32)

    lab_f = labels.astype(jnp.float32)
    boxes = pl.pallas_call(
        _boxes_body,
        grid=(n // _BNB,),
        in_specs=[
            pl.BlockSpec((_BNB, h, w), lambda i: (i, 0, 0)),
            pl.BlockSpec((_BNB, 1), lambda i: (i, 0)),
        ],
        out_specs=pl.BlockSpec((_BNB, 4), lambda i: (i, 0)),
        out_shape=jax.ShapeDtypeStruct((n, 4), jnp.float32),
        compiler_params=pltpu.CompilerParams(
            dimension_semantics=("arbitrary",)),
    )(masks, lab_f.reshape(n, 1))

    keep, kept_points = pl.pallas_call(
        _nms_body,
        out_shape=[
            jax.ShapeDtypeStruct((n, 1), jnp.float32),
            jax.ShapeDtypeStruct((n, 3), jnp.float32),
        ],
        scratch_shapes=[
            pltpu.VMEM((n, 1), jnp.float32),
            pltpu.VMEM((n, 1), jnp.float32),
            pltpu.VMEM((1, n), jnp.float32),
            pltpu.VMEM((n, 1), jnp.float32),
            pltpu.VMEM((1, n), jnp.float32),
            pltpu.VMEM((n, 1), jnp.float32),
            pltpu.VMEM((1, n), jnp.float32),
            pltpu.VMEM((4, n), jnp.float32),
        ],
    )(boxes, scores.reshape(n, 1), scores.reshape(1, n),
      foreground_points.astype(jnp.float32))

    kept_masks = pl.pallas_call(
        _apply_body,
        grid=(n // _BN,),
        in_specs=[
            pl.BlockSpec((_BN, h, w), lambda i: (i, 0, 0)),
            pl.BlockSpec((_BN, 1, 1), lambda i: (i, 0, 0)),
        ],
        out_specs=pl.BlockSpec((_BN, h, w), lambda i: (i, 0, 0)),
        out_shape=jax.ShapeDtypeStruct((n, h, w), jnp.float32),
        compiler_params=pltpu.CompilerParams(
            dimension_semantics=("arbitrary",)),
    )(masks, keep.reshape(n, 1, 1))

    return kept_masks, kept_points.astype(foreground_points.dtype)
